# Initial kernel scaffold; baseline (speedup 1.0000x reference)
import jax
import jax.numpy as jnp
from jax.experimental import pallas as pl


def kernel(x, n_data, h_mb, c_mb, e_type, W_iou, U_iou, b_iou, O_f, Z_f, Q_f, K_f, v_f, V_f, W_a, W_d):
    raise NotImplementedError("write your pallas kernel here")


# speedup vs baseline: 2.8666x; 2.8666x over previous
import functools

import jax
import jax.numpy as jnp
from jax.experimental import pallas as pl

H = 256


def _tree_lstm_block(x_ref, n_ref, h_ref, c_ref, et_ref, woz_ref, m_ref,
                     vqk_ref, b_ref, h_out, c_out):
    bb = h_ref.shape[0]
    k = h_ref.shape[1]

    h3 = h_ref[...]
    et = et_ref[...]

    h2 = h3.reshape(bb * k, H)
    oz = jnp.dot(h2, woz_ref[...], preferred_element_type=jnp.float32)
    oz3 = oz.reshape(bb, k, 2 * H)
    o3 = oz3[:, :, :H]
    z3 = oz3[:, :, H:]
    et3 = et[:, :, None]
    f3 = jax.nn.sigmoid(et3 * o3 + (1.0 - et3) * z3)
    c_red = jnp.sum(f3 * c_ref[...], axis=1)

    vk = vqk_ref[0, :]
    vq = vqk_ref[1, :]
    hv = jnp.sum(h3 * vk[None, None, :], axis=2)
    nv = jnp.sum(n_ref[...] * vq[None, :], axis=1, keepdims=True)
    logits = jnp.tanh(hv + nv)
    lmax = jnp.max(logits, axis=1, keepdims=True)
    ew = jnp.exp(logits - lmax)
    w = ew / jnp.sum(ew, axis=1, keepdims=True)

    ms = jnp.mean(et, axis=1, keepdims=True)
    wa = w * et * (1.0 - ms)
    wd = w * (1.0 - et) * ms
    s_a = jnp.sum(wa[:, :, None] * h3, axis=1)
    s_d = jnp.sum(wd[:, :, None] * h3, axis=1)

    a = jnp.concatenate([x_ref[...], s_d, s_a], axis=1)
    iou = jnp.dot(a, m_ref[...], preferred_element_type=jnp.float32) + b_ref[...]

    i_g = iou[:, :H]
    o_g = iou[:, H:2 * H]
    u_g = iou[:, 2 * H:]
    c = jax.nn.sigmoid(i_g) * jnp.tanh(u_g) + c_red
    c_out[...] = c
    h_out[...] = jax.nn.sigmoid(o_g) * jnp.tanh(c)


@functools.partial(jax.jit, static_argnames=())
def kernel(x, n_data, h_mb, c_mb, e_type, W_iou, U_iou, b_iou, O_f, Z_f, Q_f,
           K_f, v_f, V_f, W_a, W_d):
    b, k, h = h_mb.shape
    xdim = x.shape[1]

    woz = jnp.concatenate([O_f, Z_f], axis=0).T
    vk = V_f @ K_f
    vq = V_f @ Q_f
    vqk = jnp.concatenate([vk, vq], axis=0)
    fold_d = (U_iou[:, :h] @ W_d @ v_f @ K_f).T
    fold_a = (U_iou[:, h:] @ W_a @ v_f @ K_f).T
    m_mat = jnp.concatenate([W_iou.T, fold_d, fold_a], axis=0)

    bb = 200
    grid = (b // bb,)

    out_shape = (
        jax.ShapeDtypeStruct((b, h), jnp.float32),
        jax.ShapeDtypeStruct((b, h), jnp.float32),
    )
    h_out, c_out = pl.pallas_call(
        _tree_lstm_block,
        grid=grid,
        in_specs=[
            pl.BlockSpec((bb, xdim), lambda i: (i, 0)),
            pl.BlockSpec((bb, h), lambda i: (i, 0)),
            pl.BlockSpec((bb, k, h), lambda i: (i, 0, 0)),
            pl.BlockSpec((bb, k, h), lambda i: (i, 0, 0)),
            pl.BlockSpec((bb, k), lambda i: (i, 0)),
            pl.BlockSpec((h, 2 * h), lambda i: (0, 0)),
            pl.BlockSpec((xdim + 2 * h, 3 * h), lambda i: (0, 0)),
            pl.BlockSpec((2, h), lambda i: (0, 0)),
            pl.BlockSpec((1, 3 * h), lambda i: (0, 0)),
        ],
        out_specs=(
            pl.BlockSpec((bb, h), lambda i: (i, 0)),
            pl.BlockSpec((bb, h), lambda i: (i, 0)),
        ),
        out_shape=out_shape,
    )(x, n_data, h_mb, c_mb, e_type, woz, m_mat, vqk, b_iou)
    return h_out, c_out


# speedup vs baseline: 3.9635x; 1.3827x over previous
import functools

import jax
import jax.numpy as jnp
from jax.experimental import pallas as pl

H = 256
R = 128


def _tree_lstm_block(x_ref, n_ref, h_ref, c_ref, et_ref, woz_ref, vq_ref,
                     m_ref, b_ref, h_out, c_out):
    bb = h_ref.shape[0]
    k = h_ref.shape[1]

    h3 = h_ref[...]
    et = et_ref[...]

    h2 = h3.reshape(bb * k, H)
    ozv = jnp.dot(h2, woz_ref[...], preferred_element_type=jnp.float32)
    ozv3 = ozv.reshape(bb, k, 2 * H + R)
    d3 = ozv3[:, :, :H]
    z3 = ozv3[:, :, H:2 * H]
    hv = ozv3[:, :, 2 * H:]

    nv = jnp.dot(n_ref[...], vq_ref[...], preferred_element_type=jnp.float32)

    ew = jnp.exp(jnp.tanh(hv + nv[:, None, :]))
    w128 = ew / jnp.sum(ew, axis=1, keepdims=True)
    w256 = jnp.concatenate([w128, w128], axis=2)

    et3 = et[:, :, None]
    f3 = jax.nn.sigmoid(z3 + et3 * d3)
    c_red = jnp.sum(f3 * c_ref[...], axis=1)

    wa3 = w256 * et3
    wd3 = w256 - wa3
    s_a = jnp.sum(wa3 * h3, axis=1)
    s_d = jnp.sum(wd3 * h3, axis=1)

    ms = jnp.sum(et, axis=1, keepdims=True) * (1.0 / k)
    a = jnp.concatenate([x_ref[...], ms * s_d, (1.0 - ms) * s_a], axis=1)
    iou = jnp.dot(a, m_ref[...], preferred_element_type=jnp.float32) + b_ref[...]

    i_g = iou[:, :H]
    o_g = iou[:, H:2 * H]
    u_g = iou[:, 2 * H:]
    c = jax.nn.sigmoid(i_g) * jnp.tanh(u_g) + c_red
    c_out[...] = c
    h_out[...] = jax.nn.sigmoid(o_g) * jnp.tanh(c)


@functools.partial(jax.jit, static_argnames=())
def kernel(x, n_data, h_mb, c_mb, e_type, W_iou, U_iou, b_iou, O_f, Z_f, Q_f,
           K_f, v_f, V_f, W_a, W_d):
    b, k, h = h_mb.shape
    xdim = x.shape[1]

    vk = (V_f @ K_f).T
    vq = (V_f @ Q_f).T
    ones_r = jnp.ones((1, R), jnp.float32)
    woz = jnp.concatenate(
        [(O_f - Z_f).T, Z_f.T, vk @ ones_r], axis=1)
    vq_rep = vq @ ones_r
    fold_d = (U_iou[:, :h] @ W_d @ v_f @ K_f).T
    fold_a = (U_iou[:, h:] @ W_a @ v_f @ K_f).T
    m_mat = jnp.concatenate([W_iou.T, fold_d, fold_a], axis=0)

    bb = 200
    grid = (b // bb,)

    out_shape = (
        jax.ShapeDtypeStruct((b, h), jnp.float32),
        jax.ShapeDtypeStruct((b, h), jnp.float32),
    )
    h_out, c_out = pl.pallas_call(
        _tree_lstm_block,
        grid=grid,
        in_specs=[
            pl.BlockSpec((bb, xdim), lambda i: (i, 0)),
            pl.BlockSpec((bb, h), lambda i: (i, 0)),
            pl.BlockSpec((bb, k, h), lambda i: (i, 0, 0)),
            pl.BlockSpec((bb, k, h), lambda i: (i, 0, 0)),
            pl.BlockSpec((bb, k), lambda i: (i, 0)),
            pl.BlockSpec((h, 2 * h + R), lambda i: (0, 0)),
            pl.BlockSpec((h, R), lambda i: (0, 0)),
            pl.BlockSpec((xdim + 2 * h, 3 * h), lambda i: (0, 0)),
            pl.BlockSpec((1, 3 * h), lambda i: (0, 0)),
        ],
        out_specs=(
            pl.BlockSpec((bb, h), lambda i: (i, 0)),
            pl.BlockSpec((bb, h), lambda i: (i, 0)),
        ),
        out_shape=out_shape,
    )(x, n_data, h_mb, c_mb, e_type, woz, vq_rep, m_mat, b_iou)
    return h_out, c_out


# speedup vs baseline: 4.0207x; 1.0144x over previous
import functools

import jax
import jax.numpy as jnp
from jax.experimental import pallas as pl

H = 256
R = 128


def _tree_lstm_block(x_ref, n_ref, h_ref, c_ref, et_ref, woz_ref, vq_ref,
                     m_ref, b_ref, h_out, c_out):
    bb = h_ref.shape[0]
    k = h_ref.shape[1]

    h3 = h_ref[...]
    et = et_ref[...]

    h2 = h3.reshape(bb * k, H)
    ozv = jnp.dot(h2, woz_ref[...], preferred_element_type=jnp.float32)
    ozv3 = ozv.reshape(bb, k, 2 * H + R)
    d3 = ozv3[:, :, :H]
    z3 = ozv3[:, :, H:2 * H]
    hv = ozv3[:, :, 2 * H:]

    nv = jnp.dot(n_ref[...], vq_ref[...], preferred_element_type=jnp.float32)

    ew = jnp.exp(jnp.tanh(hv + nv[:, None, :]))
    w128 = ew * (1.0 / jnp.sum(ew, axis=1, keepdims=True))

    et3 = et[:, :, None]
    f3 = jax.nn.sigmoid(z3 + et3 * d3)
    c_red = jnp.sum(f3 * c_ref[...], axis=1)

    g_lo = w128 * h3[:, :, :R]
    g_hi = w128 * h3[:, :, R:]
    s_w = jnp.concatenate(
        [jnp.sum(g_lo, axis=1), jnp.sum(g_hi, axis=1)], axis=1)
    s_a = jnp.concatenate(
        [jnp.sum(g_lo * et3, axis=1), jnp.sum(g_hi * et3, axis=1)], axis=1)
    s_d = s_w - s_a

    ms = jnp.sum(et, axis=1, keepdims=True) * (1.0 / k)
    a = jnp.concatenate([x_ref[...], ms * s_d, (1.0 - ms) * s_a], axis=1)
    iou = jnp.dot(a, m_ref[...], preferred_element_type=jnp.float32) + b_ref[...]

    i_g = iou[:, :H]
    o_g = iou[:, H:2 * H]
    u_g = iou[:, 2 * H:]
    c = jax.nn.sigmoid(i_g) * jnp.tanh(u_g) + c_red
    c_out[...] = c
    h_out[...] = jax.nn.sigmoid(o_g) * jnp.tanh(c)


@functools.partial(jax.jit, static_argnames=())
def kernel(x, n_data, h_mb, c_mb, e_type, W_iou, U_iou, b_iou, O_f, Z_f, Q_f,
           K_f, v_f, V_f, W_a, W_d):
    b, k, h = h_mb.shape
    xdim = x.shape[1]

    vk = (V_f @ K_f).T
    vq = (V_f @ Q_f).T
    ones_r = jnp.ones((1, R), jnp.float32)
    woz = jnp.concatenate(
        [(O_f - Z_f).T, Z_f.T, vk @ ones_r], axis=1)
    vq_rep = vq @ ones_r
    fold_d = (U_iou[:, :h] @ W_d @ v_f @ K_f).T
    fold_a = (U_iou[:, h:] @ W_a @ v_f @ K_f).T
    m_mat = jnp.concatenate([W_iou.T, fold_d, fold_a], axis=0)

    bb = 200
    grid = (b // bb,)

    out_shape = (
        jax.ShapeDtypeStruct((b, h), jnp.float32),
        jax.ShapeDtypeStruct((b, h), jnp.float32),
    )
    h_out, c_out = pl.pallas_call(
        _tree_lstm_block,
        grid=grid,
        in_specs=[
            pl.BlockSpec((bb, xdim), lambda i: (i, 0)),
            pl.BlockSpec((bb, h), lambda i: (i, 0)),
            pl.BlockSpec((bb, k, h), lambda i: (i, 0, 0)),
            pl.BlockSpec((bb, k, h), lambda i: (i, 0, 0)),
            pl.BlockSpec((bb, k), lambda i: (i, 0)),
            pl.BlockSpec((h, 2 * h + R), lambda i: (0, 0)),
            pl.BlockSpec((h, R), lambda i: (0, 0)),
            pl.BlockSpec((xdim + 2 * h, 3 * h), lambda i: (0, 0)),
            pl.BlockSpec((1, 3 * h), lambda i: (0, 0)),
        ],
        out_specs=(
            pl.BlockSpec((bb, h), lambda i: (i, 0)),
            pl.BlockSpec((bb, h), lambda i: (i, 0)),
        ),
        out_shape=out_shape,
    )(x, n_data, h_mb, c_mb, e_type, woz, vq_rep, m_mat, b_iou)
    return h_out, c_out


# speedup vs baseline: 4.2115x; 1.0475x over previous
import functools

import jax
import jax.numpy as jnp
from jax.experimental import pallas as pl

H = 256
R = 128


def _tree_lstm_block(x_ref, n_ref, h_ref, c_ref, et_ref, woz_ref, vq_ref,
                     m_ref, b_ref, h_out, c_out):
    bb = h_ref.shape[0]
    k = h_ref.shape[1]

    h3 = h_ref[...]
    et = et_ref[...]

    h2 = h3.reshape(bb * k, H)
    ozv = jnp.dot(h2, woz_ref[...], preferred_element_type=jnp.float32)
    ozv3 = ozv.reshape(bb, k, 2 * H + R)
    d3 = ozv3[:, :, :H]
    z3 = ozv3[:, :, H:2 * H]
    hv = ozv3[:, :, 2 * H:]

    nv = jnp.dot(n_ref[...], vq_ref[...], preferred_element_type=jnp.float32)

    ew = jnp.exp(jnp.tanh(hv + nv[:, None, :]))
    w128 = ew * (1.0 / jnp.sum(ew, axis=1, keepdims=True))

    et3 = et[:, :, None]
    f3 = jax.nn.sigmoid(z3 + et3 * d3)
    c_red = jnp.sum(f3 * c_ref[...], axis=1)

    g_lo = w128 * h3[:, :, :R]
    g_hi = w128 * h3[:, :, R:]
    s_w = jnp.concatenate(
        [jnp.sum(g_lo, axis=1), jnp.sum(g_hi, axis=1)], axis=1)
    s_a = jnp.concatenate(
        [jnp.sum(g_lo * et3, axis=1), jnp.sum(g_hi * et3, axis=1)], axis=1)
    s_d = s_w - s_a

    ms = jnp.sum(et, axis=1, keepdims=True) * (1.0 / k)
    a = jnp.concatenate([x_ref[...], ms * s_d, (1.0 - ms) * s_a], axis=1)
    iou = jnp.dot(a, m_ref[...], preferred_element_type=jnp.float32) + b_ref[...]

    i_g = iou[:, :H]
    o_g = iou[:, H:2 * H]
    u_g = iou[:, 2 * H:]
    c = jax.nn.sigmoid(i_g) * jnp.tanh(u_g) + c_red
    c_out[...] = c
    h_out[...] = jax.nn.sigmoid(o_g) * jnp.tanh(c)


@functools.partial(jax.jit, static_argnames=())
def kernel(x, n_data, h_mb, c_mb, e_type, W_iou, U_iou, b_iou, O_f, Z_f, Q_f,
           K_f, v_f, V_f, W_a, W_d):
    b, k, h = h_mb.shape
    xdim = x.shape[1]

    vk = (V_f @ K_f).T
    vq = (V_f @ Q_f).T
    ones_r = jnp.ones((1, R), jnp.float32)
    woz = jnp.concatenate(
        [(O_f - Z_f).T, Z_f.T, vk @ ones_r], axis=1)
    vq_rep = vq @ ones_r
    fold_d = (U_iou[:, :h] @ W_d @ v_f @ K_f).T
    fold_a = (U_iou[:, h:] @ W_a @ v_f @ K_f).T
    m_mat = jnp.concatenate([W_iou.T, fold_d, fold_a], axis=0)

    bb = 400
    grid = (b // bb,)

    out_shape = (
        jax.ShapeDtypeStruct((b, h), jnp.float32),
        jax.ShapeDtypeStruct((b, h), jnp.float32),
    )
    h_out, c_out = pl.pallas_call(
        _tree_lstm_block,
        grid=grid,
        in_specs=[
            pl.BlockSpec((bb, xdim), lambda i: (i, 0)),
            pl.BlockSpec((bb, h), lambda i: (i, 0)),
            pl.BlockSpec((bb, k, h), lambda i: (i, 0, 0)),
            pl.BlockSpec((bb, k, h), lambda i: (i, 0, 0)),
            pl.BlockSpec((bb, k), lambda i: (i, 0)),
            pl.BlockSpec((h, 2 * h + R), lambda i: (0, 0)),
            pl.BlockSpec((h, R), lambda i: (0, 0)),
            pl.BlockSpec((xdim + 2 * h, 3 * h), lambda i: (0, 0)),
            pl.BlockSpec((1, 3 * h), lambda i: (0, 0)),
        ],
        out_specs=(
            pl.BlockSpec((bb, h), lambda i: (i, 0)),
            pl.BlockSpec((bb, h), lambda i: (i, 0)),
        ),
        out_shape=out_shape,
    )(x, n_data, h_mb, c_mb, e_type, woz, vq_rep, m_mat, b_iou)
    return h_out, c_out
